# Initial kernel scaffold; baseline (speedup 1.0000x reference)
import jax
import jax.numpy as jnp
from jax.experimental import pallas as pl


def kernel(feat, edge_index, emb, bias):
    raise NotImplementedError("write your pallas kernel here")


# speedup vs baseline: 5.5916x; 5.5916x over previous
import functools

import jax
import jax.numpy as jnp
from jax import lax
from jax.experimental import pallas as pl
from jax.experimental.pallas import tpu as pltpu
from jax.experimental.pallas import tpu_sc as plsc

N_NODES = 10000
N_EDGES = 320000
IN_FEATS = 10000
OUT_FEATS = 128

NC = 2
NS = 16
L = 16
FH = OUT_FEATS // NC

NPT = 640
N_PAD = NS * NPT
NODE_CHUNKS = NPT // 128

EDGE_CHUNK = 128
GCHUNK = 16
GROUPS = 10
CHUNKS = GROUPS * GCHUNK
EPT = CHUNKS * EDGE_CHUNK
E_PAD = NS * EPT


def _rsqrt_inplace(ref, n_vecs):
    def body(i, carry):
        x = jnp.maximum(ref[pl.ds(i * L, L)], 1.0)
        bits = lax.bitcast_convert_type(x, jnp.int32)
        y = lax.bitcast_convert_type(
            jnp.int32(0x5F3759DF) - lax.shift_right_arithmetic(bits, 1),
            jnp.float32)
        for _ in range(3):
            y = y * (1.5 - 0.5 * x * y * y)
        ref[pl.ds(i * L, L)] = y
        return carry

    lax.fori_loop(0, n_vecs, body, 0)


def _sc_body(feat_ref, src_ref, dst_ref, emb2_ref, bias_ref, out_ref,
             x_sp, accum, outdeg, indeg,
             sbuf, dbuf, featbuf, norm_v, rows0, ones_v, biasv):
    c = lax.axis_index("c")
    s = lax.axis_index("s")
    base_n = s * NPT

    zeros16 = jnp.zeros((L,), jnp.float32)

    def zero_rows(r, carry):
        for f in range(FH // L):
            rows0[r, pl.ds(f * L, L)] = zeros16
        return carry

    lax.fori_loop(0, 128, zero_rows, 0)

    def zero_norm(i, carry):
        norm_v[pl.ds(i * L, L)] = zeros16
        return carry

    lax.fori_loop(0, NPT // L, zero_norm, 0)

    ones16 = jnp.ones((L,), jnp.float32)
    for i in range(EDGE_CHUNK // L):
        ones_v[pl.ds(i * L, L)] = ones16

    for j in range(NODE_CHUNKS):
        pltpu.sync_copy(rows0, accum.at[pl.ds(base_n + j * 128, 128)])
    pltpu.sync_copy(norm_v, outdeg.at[pl.ds(base_n, NPT)])
    pltpu.sync_copy(norm_v, indeg.at[pl.ds(base_n, NPT)])

    pltpu.sync_copy(feat_ref.at[pl.ds(base_n, NPT)], featbuf)
    pltpu.sync_copy(bias_ref.at[pl.ds(c * FH, FH)], biasv)

    coff = (c * IN_FEATS).astype(jnp.int32)

    def add_off(i, carry):
        featbuf[pl.ds(i * L, L)] = featbuf[pl.ds(i * L, L)] + coff
        return carry

    lax.fori_loop(0, NPT // L, add_off, 0)

    plsc.subcore_barrier()

    def hist_group(g, carry):
        pltpu.sync_copy(src_ref.at[s, pl.ds(g * GCHUNK, GCHUNK)], sbuf)
        pltpu.sync_copy(dst_ref.at[s, pl.ds(g * GCHUNK, GCHUNK)], dbuf)

        def hist(k, carry2):
            pltpu.sync_copy(ones_v, outdeg.at[sbuf.at[k]], add=True)
            pltpu.sync_copy(ones_v, indeg.at[dbuf.at[k]], add=True)
            return carry2

        lax.fori_loop(0, GCHUNK, hist, 0)
        return carry

    lax.fori_loop(0, GROUPS, hist_group, 0)
    plsc.subcore_barrier()

    pltpu.sync_copy(outdeg.at[pl.ds(base_n, NPT)], norm_v)
    _rsqrt_inplace(norm_v, NPT // L)

    lane_iota = lax.iota(jnp.int32, L)

    def scale_rows(j, bias_vecs=None):
        def group(g, carry2):
            nv16 = norm_v[pl.ds(j * 128 + g * L, L)]
            for r16 in range(L):
                bc = jnp.full((L,), jnp.sum(jnp.where(lane_iota == r16,
                                                      nv16, 0.0)))
                r = g * L + r16
                for f in range(FH // L):
                    v = rows0[r, pl.ds(f * L, L)] * bc
                    if bias_vecs is not None:
                        v = v + bias_vecs[f]
                    rows0[r, pl.ds(f * L, L)] = v
            return carry2

        lax.fori_loop(0, 128 // L, group, 0)

    def build_chunk(j, carry):
        pltpu.sync_copy(emb2_ref.at[featbuf.at[pl.ds(j * 128, 128)]], rows0)
        scale_rows(j)
        pltpu.sync_copy(rows0, x_sp.at[pl.ds(base_n + j * 128, 128)])
        return carry

    lax.fori_loop(0, NODE_CHUNKS, build_chunk, 0)

    pltpu.sync_copy(indeg.at[pl.ds(base_n, NPT)], norm_v)
    _rsqrt_inplace(norm_v, NPT // L)
    plsc.subcore_barrier()

    def edge_group(g, carry):
        pltpu.sync_copy(src_ref.at[s, pl.ds(g * GCHUNK, GCHUNK)], sbuf)
        pltpu.sync_copy(dst_ref.at[s, pl.ds(g * GCHUNK, GCHUNK)], dbuf)

        def edges(k, carry2):
            pltpu.sync_copy(x_sp.at[sbuf.at[k]], rows0)
            pltpu.sync_copy(rows0, accum.at[dbuf.at[k]], add=True)
            return carry2

        lax.fori_loop(0, GCHUNK, edges, 0)
        return carry

    lax.fori_loop(0, GROUPS, edge_group, 0)
    plsc.subcore_barrier()

    bias_vecs = [biasv[pl.ds(f * L, L)] for f in range(FH // L)]

    def out_chunk(j, carry):
        pltpu.sync_copy(accum.at[pl.ds(base_n + j * 128, 128)], rows0)
        scale_rows(j, bias_vecs)
        pltpu.sync_copy(rows0,
                        out_ref.at[c, pl.ds(base_n + j * 128, 128)])
        return carry

    lax.fori_loop(0, NODE_CHUNKS, out_chunk, 0)


@functools.partial(jax.jit, static_argnames=())
def kernel(feat, edge_index, emb, bias):
    feat = feat.astype(jnp.int32)
    src = edge_index[0].astype(jnp.int32)
    dst = edge_index[1].astype(jnp.int32)

    emb2 = jnp.concatenate([emb[:, :FH], emb[:, FH:]], axis=0)

    feat_p = jnp.concatenate(
        [feat, jnp.zeros((N_PAD - N_NODES,), jnp.int32)])
    npad = E_PAD - N_EDGES
    pad_idx = (jnp.arange(npad, dtype=jnp.int32) % (N_PAD - N_NODES)
               ) + N_NODES
    src_p = jnp.concatenate([src, pad_idx]).reshape(NS, CHUNKS, EDGE_CHUNK)
    dst_p = jnp.concatenate([dst, pad_idx]).reshape(NS, CHUNKS, EDGE_CHUNK)

    mesh = plsc.VectorSubcoreMesh(core_axis_name="c", subcore_axis_name="s",
                                  num_cores=NC, num_subcores=NS)
    out = pl.kernel(
        _sc_body,
        out_type=jax.ShapeDtypeStruct((NC, N_PAD, FH), jnp.float32),
        mesh=mesh,
        compiler_params=pltpu.CompilerParams(needs_layout_passes=False,
                                             use_tc_tiling_on_sc=False),
        scratch_types=[
            pltpu.VMEM_SHARED((N_PAD, FH), jnp.float32),
            pltpu.VMEM_SHARED((N_PAD, FH), jnp.float32),
            pltpu.VMEM_SHARED((N_PAD,), jnp.float32),
            pltpu.VMEM_SHARED((N_PAD,), jnp.float32),
            pltpu.VMEM((GCHUNK, EDGE_CHUNK), jnp.int32),
            pltpu.VMEM((GCHUNK, EDGE_CHUNK), jnp.int32),
            pltpu.VMEM((NPT,), jnp.int32),
            pltpu.VMEM((NPT,), jnp.float32),
            pltpu.VMEM((128, FH), jnp.float32),
            pltpu.VMEM((EDGE_CHUNK,), jnp.float32),
            pltpu.VMEM((FH,), jnp.float32),
        ],
    )(feat_p, src_p, dst_p, emb2, bias)
    return jnp.concatenate([out[0, :N_NODES], out[1, :N_NODES]], axis=1)
